# baseline (speedup 1.0000x reference)
import dataclasses
import functools

import jax
import jax.numpy as jnp
from jax import lax
from jax.experimental import pallas as pl
from jax.experimental.pallas import tpu as pltpu
from jax.experimental.pallas import tpu_sc as plsc

N0, N1, N2 = 50000, 12500, 3125
E0, E1, E2 = 800000, 200000, 50000
LATENT = 128

NC, NS, NLANE = 2, 16, 16
NW = NC * NS
_CHUNK = 128
_BIG = 256


def _leaky(x):
    return jnp.where(x > 0, x, 0.01 * x)


def _rup(n, m):
    return ((n + m - 1) // m) * m


def _tc_map(fn, out_rows, out_dim, row_ins, const_ins, blk=1024):
    nr = len(row_ins)

    def body(*refs):
        rows = [r[...] for r in refs[:nr]]
        cs = [r[...] for r in refs[nr:-1]]
        refs[-1][...] = fn(pl.program_id(0), rows, cs)

    in_specs = [pl.BlockSpec((blk, a.shape[1]), lambda i: (i, 0)) for a in row_ins]
    for c in const_ins:
        nd = c.ndim
        in_specs.append(pl.BlockSpec(c.shape, (lambda i, _nd=nd: (0,) * _nd)))
    return pl.pallas_call(
        body,
        grid=(pl.cdiv(out_rows, blk),),
        in_specs=in_specs,
        out_specs=pl.BlockSpec((blk, out_dim), lambda i: (i, 0)),
        out_shape=jax.ShapeDtypeStruct((out_rows, out_dim), jnp.float32),
    )(*row_ins, *const_ins)


def _dot(a, w):
    return jnp.dot(a, w, preferred_element_type=jnp.float32)


def _proj_pad(x, w, n_valid=None, blk=1024):
    n = x.shape[0] if n_valid is None else n_valid

    def fn(i, rows, cs):
        y = _dot(rows[0], cs[0])
        rid = i * blk + lax.broadcasted_iota(jnp.int32, y.shape, 0)
        return jnp.where(rid < n, y, 0.0)

    assert n % blk != 0
    return _tc_map(fn, n + 1, w.shape[1], [x], [w], blk=blk)


def _tc_edge(g1, g2, ea, wa, we2, out_rows):
    def fn(i, rows, cs):
        return _dot(_leaky(rows[0] + rows[1] + _dot(rows[2], cs[0])), cs[1])
    return _tc_map(fn, out_rows, we2.shape[1], [g1, g2, ea], [wa, we2])


def _tc_edge3(g1, g2, g3, we2, out_rows):
    def fn(i, rows, cs):
        return _dot(_leaky(rows[0] + rows[1] + rows[2]), cs[0])
    return _tc_map(fn, out_rows, we2.shape[1], [g1, g2, g3], [we2])


def _tc_leaky3(g1, g2, g3, out_rows):
    def fn(i, rows, cs):
        return _leaky(rows[0] + rows[1] + rows[2])
    return _tc_map(fn, out_rows, g1.shape[1], [g1, g2, g3], [])


def _tc_node_direct(x, agg, wn1x, wn1g, wn2):
    def fn(i, rows, cs):
        return _dot(_leaky(_dot(rows[0], cs[0]) + _dot(rows[1], cs[1])), cs[2])
    return _tc_map(fn, x.shape[0], wn2.shape[1], [x, agg], [wn1x, wn1g, wn2])


def _tc_node_skip(xn, aggs, we2s, wn1gs, wn2s, out_rows):
    def fn(i, rows, cs):
        return _dot(_leaky(rows[0] + _dot(_dot(rows[1], cs[0]), cs[1])), cs[2])
    return _tc_map(fn, out_rows, wn2s.shape[1], [xn, aggs], [we2s, wn1gs, wn2s])


def _tc_node_main(xn, agg, xs, wn1g, wn2, out_rows):
    def fn(i, rows, cs):
        return _leaky(_dot(_leaky(rows[0] + _dot(rows[1], cs[0])), cs[1]) + rows[2])
    return _tc_map(fn, out_rows, wn2.shape[1], [xn, agg, xs], [wn1g, wn2])


def _tc_head(x, p, out_rows):
    w1, b1 = p['W1'], p['b1'].reshape(1, -1)
    w2, b2 = p['W2'], p['b2'].reshape(1, -1)
    g, b = p['g'].reshape(1, -1), p['b'].reshape(1, -1)

    def fn(i, rows, cs):
        w1c, b1c, w2c, b2c, gc, bc = cs
        t = _dot(_leaky(_dot(rows[0], w1c) + b1c), w2c) + b2c
        mu = jnp.mean(t, axis=-1, keepdims=True)
        var = jnp.mean((t - mu) * (t - mu), axis=-1, keepdims=True)
        return (t - mu) * jax.lax.rsqrt(var + 1e-5) * gc + bc

    return _tc_map(fn, out_rows, 3, [x], [w1, b1, w2, b2, g, b])


def _tc_latent(z2, w1, b1, w2, b2col, blk):
    L = z2.shape[0]
    K = w1.shape[1]
    n = w2.shape[1]

    def body(z_ref, w1_ref, b1_ref, w2_ref, b2_ref, o_ref):
        a = _leaky(z_ref[...] * w1_ref[...] + b1_ref[...])
        o_ref[...] = (
            lax.dot_general(w2_ref[...], a, (((0,), (1,)), ((), ())),
                            preferred_element_type=jnp.float32)
            + b2_ref[...]
        )

    return pl.pallas_call(
        body,
        grid=(pl.cdiv(n, blk),),
        in_specs=[
            pl.BlockSpec((L, 1), lambda i: (0, 0)),
            pl.BlockSpec((1, K), lambda i: (0, 0)),
            pl.BlockSpec((1, K), lambda i: (0, 0)),
            pl.BlockSpec((K, blk), lambda i: (0, i)),
            pl.BlockSpec((blk, 1), lambda i: (i, 0)),
        ],
        out_specs=pl.BlockSpec((blk, L), lambda i: (i, 0)),
        out_shape=jax.ShapeDtypeStruct((n, L), jnp.float32),
    )(z2, w1, b1.reshape(1, K), w2, b2col)


def _sc_mesh():
    return plsc.VectorSubcoreMesh(core_axis_name="c", subcore_axis_name="s")


def _sc_params(layout_passes=False):
    cp = pltpu.CompilerParams()
    fields = pltpu.CompilerParams.__dataclass_fields__
    if not layout_passes and "needs_layout_passes" in fields:
        cp = dataclasses.replace(cp, needs_layout_passes=False)
    if "use_tc_tiling_on_sc" in fields:
        cp = dataclasses.replace(cp, use_tc_tiling_on_sc=False)
    return cp


def _stride_chunks(wid, nworkers, nchunks, do):
    per = (nchunks + nworkers - 1) // nworkers

    @pl.loop(0, per)
    def _(i):
        ci = wid + i * nworkers

        @pl.when(ci < nchunks)
        def _():
            do(ci)


def _sc_translate(table, idx):
    ep = idx.shape[0]
    t = table.shape[0]
    assert ep % _CHUNK == 0
    nchunks = ep // _CHUNK

    @functools.partial(
        pl.kernel,
        out_type=jax.ShapeDtypeStruct((ep,), jnp.int32),
        mesh=_sc_mesh(),
        compiler_params=_sc_params(),
        scratch_types=[
            pltpu.VMEM((t,), jnp.int32),
            pltpu.VMEM((_CHUNK,), jnp.int32),
            pltpu.VMEM((_CHUNK,), jnp.int32),
        ],
    )
    def k(idx_hbm, tab_hbm, out_hbm, tab_v, in_v, out_v):
        wid = lax.axis_index("s") * NC + lax.axis_index("c")
        pltpu.sync_copy(tab_hbm, tab_v)

        def do(ci):
            base = ci * _CHUNK
            pltpu.sync_copy(idx_hbm.at[pl.ds(base, _CHUNK)], in_v)

            @pl.loop(0, _CHUNK, step=NLANE)
            def _(j):
                out_v[pl.ds(j, NLANE)] = plsc.load_gather(
                    tab_v, [in_v[pl.ds(j, NLANE)]])

            pltpu.sync_copy(out_v, out_hbm.at[pl.ds(base, _CHUNK)])

        _stride_chunks(wid, NW, nchunks, do)

    return k(idx, table)


def _sc_gather(table, idx):
    w = table.shape[1]
    ep = idx.shape[0]
    assert ep % _CHUNK == 0
    nchunks = ep // _CHUNK
    nbuf = min(8, max(2, (1 << 18) // (_CHUNK * w * 4)))
    per = _rup(-(-nchunks // NW), nbuf)

    scratch = [pltpu.VMEM((_CHUNK,), jnp.int32) for _ in range(nbuf)]
    scratch += [pltpu.VMEM((_CHUNK, w), jnp.float32) for _ in range(nbuf)]
    scratch += [pltpu.SemaphoreType.DMA]

    @functools.partial(
        pl.kernel,
        out_type=jax.ShapeDtypeStruct((ep, w), jnp.float32),
        mesh=_sc_mesh(),
        compiler_params=_sc_params(layout_passes=True),
        scratch_types=scratch,
    )
    def k(idx_hbm, tab_hbm, out_hbm, *rest):
        idx_v = rest[:nbuf]
        row_v = rest[nbuf:2 * nbuf]
        sem = rest[-1]
        wid = lax.axis_index("s") * NC + lax.axis_index("c")

        @pl.loop(0, per // nbuf)
        def _(gi):
            cis = [jnp.remainder(wid + (gi * nbuf + b) * NW, nchunks)
                   for b in range(nbuf)]
            hs = [pltpu.async_copy(
                idx_hbm.at[pl.ds(cis[b] * _CHUNK, _CHUNK)], idx_v[b], sem)
                for b in range(nbuf)]
            for h in hs:
                h.wait()
            hs = [pltpu.async_copy(tab_hbm.at[idx_v[b]], row_v[b], sem)
                  for b in range(nbuf)]
            for h in hs:
                h.wait()
            hs = [pltpu.async_copy(
                row_v[b], out_hbm.at[pl.ds(cis[b] * _CHUNK, _CHUNK)], sem)
                for b in range(nbuf)]
            for h in hs:
                h.wait()

    return k(idx, table)


def _vextract(ref, k):
    g = (k // NLANE) * NLANE
    v = ref[pl.ds(g, NLANE)]
    return jnp.sum(jnp.where(lax.iota(jnp.int32, NLANE) == (k - g), v, 0))


def _segsum_meta(dst_pad, n_out):
    ep = dst_pad.shape[0]
    cap = _rup(-(-n_out // NW), 8)
    ds_sorted, order = lax.sort(
        [dst_pad, jnp.arange(ep, dtype=jnp.int32)], num_keys=1)
    bounds = jnp.arange(NW + 1, dtype=jnp.int32) * cap
    elo = jnp.searchsorted(ds_sorted, bounds).astype(jnp.int32)
    c0 = elo[:NW] // _BIG
    nch = -(-(elo[1:] - c0 * _BIG) // _BIG)
    pad16 = _rup(NW, NLANE)
    c0 = jnp.pad(c0, (0, pad16 - NW))
    nch = jnp.pad(nch, (0, pad16 - NW))
    return {'order': order, 'ds': ds_sorted, 'c0': c0, 'nch': nch, 'cap': cap}


def _sc_segsum(rows, meta):
    ep, w = rows.shape
    assert ep % _CHUNK == 0
    cap = meta['cap']
    nmeta = meta['c0'].shape[0]
    rows_sorted = _sc_gather(rows, meta['order'])
    rflat = rows_sorted.reshape(ep * w)
    zeros = jnp.zeros((_BIG * w,), jnp.float32)
    acc_n = (cap + 8) * w
    zc_full, zc_tail = acc_n // (_BIG * w), acc_n % (_BIG * w)

    @functools.partial(
        pl.kernel,
        out_type=jax.ShapeDtypeStruct((NW * cap * w,), jnp.float32),
        mesh=_sc_mesh(),
        compiler_params=_sc_params(),
        scratch_types=[
            pltpu.VMEM((acc_n,), jnp.float32),
            pltpu.VMEM((nmeta,), jnp.int32),
            pltpu.VMEM((nmeta,), jnp.int32),
            pltpu.VMEM((_BIG,), jnp.int32),
            pltpu.VMEM((_BIG * w,), jnp.float32),
            pltpu.SemaphoreType.DMA,
        ],
    )
    def k(rf_hbm, ds_hbm, c0_hbm, nch_hbm, zero_hbm, out_hbm,
          acc, c0v, nchv, didx, rv, sem):
        t = lax.axis_index("s") * NC + lax.axis_index("c")
        nlo = t * cap
        pltpu.sync_copy(c0_hbm, c0v)
        pltpu.sync_copy(nch_hbm, nchv)
        e0 = _vextract(c0v, t)
        nck = _vextract(nchv, t)

        for zi in range(zc_full):
            pltpu.sync_copy(zero_hbm, acc.at[pl.ds(zi * _BIG * w, _BIG * w)])
        if zc_tail:
            pltpu.sync_copy(zero_hbm.at[pl.ds(0, zc_tail)],
                            acc.at[pl.ds(zc_full * _BIG * w, zc_tail)])

        @pl.loop(0, nck)
        def _(i):
            base = (e0 + i) * _BIG
            h1 = pltpu.async_copy(ds_hbm.at[pl.ds(base, _BIG)], didx, sem)
            h2 = pltpu.async_copy(rf_hbm.at[pl.ds(base * w, _BIG * w)], rv, sem)
            h1.wait()
            h2.wait()

            @pl.loop(0, _BIG, step=NLANE)
            def _(j):
                d = didx[pl.ds(j, NLANE)] - nlo
                ok = (d >= 0) & (d < cap)
                didx[pl.ds(j, NLANE)] = jnp.where(ok, d, cap)

            @pl.loop(0, _BIG)
            def _(r):
                d = _vextract(didx, r)
                for c in range(0, w, NLANE):
                    acc[pl.ds(d * w + c, NLANE)] = (
                        acc[pl.ds(d * w + c, NLANE)]
                        + rv[pl.ds(r * w + c, NLANE)])

        cw = cap * w
        oc_full, oc_tail = cw // (_BIG * w), cw % (_BIG * w)
        for oi in range(oc_full):
            pltpu.sync_copy(acc.at[pl.ds(oi * _BIG * w, _BIG * w)],
                            out_hbm.at[pl.ds(nlo * w + oi * _BIG * w, _BIG * w)])
        if oc_tail:
            pltpu.sync_copy(acc.at[pl.ds(oc_full * _BIG * w, oc_tail)],
                            out_hbm.at[pl.ds(nlo * w + oc_full * _BIG * w, oc_tail)])

    out = k(rflat, meta['ds'], meta['c0'], meta['nch'], zeros)
    return out.reshape(NW * cap, w)


def _split_mpl(p, din):
    we1, wn1 = p['We1'], p['Wn1']
    return (we1[:din], we1[din:2 * din], we1[2 * din:],
            wn1[:din], wn1[din:], p['We2'], p['Wn2'])


def _mpl_direct(x, srcp, dstp, meta, ea, p, din, n):
    ws, wd, wa, wn1x, wn1g, we2, wn2 = _split_mpl(p, din)
    ps = _proj_pad(x, ws)
    pd = _proj_pad(x, wd)
    g1 = _sc_gather(ps, srcp)
    g2 = _sc_gather(pd, dstp)
    e_out = _tc_edge(g1, g2, ea, wa, we2, g1.shape[0])
    agg = _sc_segsum(e_out, meta)
    xo = _tc_node_direct(x, agg, wn1x, wn1g, wn2)
    return xo, e_out


def _res_up(x, ea, srcp_c, dstp_c, meta_c, srcp_f, dstp_f, meta_f,
            minv, einv, n_c, n_f, e_c, p, din1, dins):
    xm, eam = _mpl_direct(x, srcp_c, dstp_c, meta_c, ea, p['mpl1'], din1, n_c)
    dout1 = p['mpl1']['We2'].shape[1]

    fsrc = _sc_translate(minv, srcp_f)
    fdst = _sc_translate(minv, dstp_f)

    q = p['mpl2']
    ws2, wd2, wa2, wn1x2, wn1g2, we2_2, wn2_2 = _split_mpl(q, dout1)
    ps2 = _proj_pad(xm, ws2)
    pd2 = _proj_pad(xm, wd2)
    ep2 = _proj_pad(eam, wa2, n_valid=e_c)
    ga = _sc_gather(ps2, fsrc)
    gb = _sc_gather(pd2, fdst)
    gc = _sc_gather(ep2, einv)
    e2 = _tc_edge3(ga, gb, gc, we2_2, ga.shape[0])
    agg2 = _sc_segsum(e2, meta_f)
    xn2 = _sc_gather(_proj_pad(xm, wn1x2), minv)

    s = p['skip']
    wss, wds, was, wn1xs, wn1gs, we2s, wn2s = _split_mpl(s, dins)
    pss = _proj_pad(x, wss)
    pds = _proj_pad(x, wds)
    eps = _proj_pad(ea, was, n_valid=e_c)
    sa = _sc_gather(pss, fsrc)
    sb = _sc_gather(pds, fdst)
    sc = _sc_gather(eps, einv)
    gs = _tc_leaky3(sa, sb, sc, sa.shape[0])
    aggs = _sc_segsum(gs, meta_f)
    xns = _sc_gather(_proj_pad(x, wn1xs), minv)
    xs = _tc_node_skip(xns, aggs, we2s, wn1gs, wn2s, n_f)

    xo = _tc_node_main(xn2, agg2, xs, wn1g2, wn2_2, n_f)
    return xo, e2


def kernel(z, edge_index_l0, edge_index_l1, edge_index_l2, m_id_0, m_id_1,
           e_idx_0, e_idx_1, params):
    i32 = jnp.int32
    ei0 = edge_index_l0.astype(i32)
    ei1 = edge_index_l1.astype(i32)
    ei2 = edge_index_l2.astype(i32)

    e0p, e1p, e2p = _rup(E0, _BIG), _rup(E1, _BIG), _rup(E2, _BIG)
    n0p, n1p = _rup(N0, _CHUNK), _rup(N1, _CHUNK)

    def pad_idx(a, ep, fill):
        return jnp.pad(a, (0, ep - a.shape[0]), constant_values=fill)

    src0, dst0 = pad_idx(ei0[0], e0p, 0), pad_idx(ei0[1], e0p, 0)
    src1, dst1 = pad_idx(ei1[0], e1p, 0), pad_idx(ei1[1], e1p, 0)
    src2, dst2 = pad_idx(ei2[0], e2p, 0), pad_idx(ei2[1], e2p, 0)
    meta0 = _segsum_meta(pad_idx(ei0[1], e0p, N0), N0)
    meta1 = _segsum_meta(pad_idx(ei1[1], e1p, N1), N1)
    meta2 = _segsum_meta(pad_idx(ei2[1], e2p, N2), N2)

    minv0 = jnp.full((n0p,), N1, i32).at[m_id_0.astype(i32)].set(
        jnp.arange(N1, dtype=i32))
    minv1 = jnp.full((n1p,), N2, i32).at[m_id_1.astype(i32)].set(
        jnp.arange(N2, dtype=i32))
    einv0 = jnp.full((e0p,), E1, i32).at[e_idx_0.astype(i32)].set(
        jnp.arange(E1, dtype=i32))
    einv1 = jnp.full((e1p,), E2, i32).at[e_idx_1.astype(i32)].set(
        jnp.arange(E2, dtype=i32))

    z2 = z.reshape(LATENT, 1)
    px, pe = params['up_x'], params['up_e']
    x = _tc_latent(z2, px['W1'], px['b1'], px['W2'],
                   px['b2'].reshape(-1, 1), blk=512)
    e = _tc_latent(z2, pe['W1'], pe['b1'], pe['W2'],
                   pe['b2'].reshape(-1, 1), blk=1024)

    x, e = _mpl_direct(x, src2, dst2, meta2, e, params['bottom'], LATENT, N2)

    x, e = _res_up(x, e, src2, dst2, meta2, src1, dst1, meta1, minv1, einv1,
                   N2, N1, E2, params['l0'], 256, 256)

    x, e = _res_up(x, e, src1, dst1, meta1, src0, dst0, meta0, minv0, einv0,
                   N1, N0, E1, params['l1'], 128, 128)

    x, e = _mpl_direct(x, src0, dst0, meta0, e, params['final'], 64, N0)

    xo = _tc_head(x, params['out_n'], N0)
    eo = _tc_head(e[:E0], params['out_e'], E0)
    return xo, eo


# speedup vs baseline: 3.4574x; 3.4574x over previous
import dataclasses
import functools

import jax
import jax.numpy as jnp
from jax import lax
from jax.experimental import pallas as pl
from jax.experimental.pallas import tpu as pltpu
from jax.experimental.pallas import tpu_sc as plsc

N0, N1, N2 = 50000, 12500, 3125
E0, E1, E2 = 800000, 200000, 50000
LATENT = 128

NC, NS, NLANE = 2, 16, 16
NW = NC * NS
_CHUNK = 128
_BIG = 256


def _leaky(x):
    return jnp.where(x > 0, x, 0.01 * x)


def _rup(n, m):
    return ((n + m - 1) // m) * m


def _tc_map(fn, out_rows, out_dim, row_ins, const_ins, blk=1024):
    nr = len(row_ins)

    def body(*refs):
        rows = [r[...] for r in refs[:nr]]
        cs = [r[...] for r in refs[nr:-1]]
        refs[-1][...] = fn(pl.program_id(0), rows, cs)

    in_specs = [pl.BlockSpec((blk, a.shape[1]), lambda i: (i, 0)) for a in row_ins]
    for c in const_ins:
        nd = c.ndim
        in_specs.append(pl.BlockSpec(c.shape, (lambda i, _nd=nd: (0,) * _nd)))
    return pl.pallas_call(
        body,
        grid=(pl.cdiv(out_rows, blk),),
        in_specs=in_specs,
        out_specs=pl.BlockSpec((blk, out_dim), lambda i: (i, 0)),
        out_shape=jax.ShapeDtypeStruct((out_rows, out_dim), jnp.float32),
    )(*row_ins, *const_ins)


def _dot(a, w):
    return jnp.dot(a, w, preferred_element_type=jnp.float32)


def _proj_pad(x, w, n_valid=None, blk=1024):
    n = x.shape[0] if n_valid is None else n_valid

    def fn(i, rows, cs):
        y = _dot(rows[0], cs[0])
        rid = i * blk + lax.broadcasted_iota(jnp.int32, y.shape, 0)
        return jnp.where(rid < n, y, 0.0)

    assert n % blk != 0
    return _tc_map(fn, n + 1, w.shape[1], [x], [w], blk=blk)


def _tc_edge(g1, g2, ea, wa, we2, out_rows):
    def fn(i, rows, cs):
        return _dot(_leaky(rows[0] + rows[1] + _dot(rows[2], cs[0])), cs[1])
    return _tc_map(fn, out_rows, we2.shape[1], [g1, g2, ea], [wa, we2])


def _tc_edge3(g1, g2, g3, we2, out_rows):
    def fn(i, rows, cs):
        return _dot(_leaky(rows[0] + rows[1] + rows[2]), cs[0])
    return _tc_map(fn, out_rows, we2.shape[1], [g1, g2, g3], [we2])


def _tc_leaky3(g1, g2, g3, out_rows):
    def fn(i, rows, cs):
        return _leaky(rows[0] + rows[1] + rows[2])
    return _tc_map(fn, out_rows, g1.shape[1], [g1, g2, g3], [])


def _tc_node_direct(x, agg, wn1x, wn1g, wn2):
    def fn(i, rows, cs):
        return _dot(_leaky(_dot(rows[0], cs[0]) + _dot(rows[1], cs[1])), cs[2])
    return _tc_map(fn, x.shape[0], wn2.shape[1], [x, agg], [wn1x, wn1g, wn2])


def _tc_node_skip(xn, aggs, we2s, wn1gs, wn2s, out_rows):
    def fn(i, rows, cs):
        return _dot(_leaky(rows[0] + _dot(_dot(rows[1], cs[0]), cs[1])), cs[2])
    return _tc_map(fn, out_rows, wn2s.shape[1], [xn, aggs], [we2s, wn1gs, wn2s])


def _tc_node_main(xn, agg, xs, wn1g, wn2, out_rows):
    def fn(i, rows, cs):
        return _leaky(_dot(_leaky(rows[0] + _dot(rows[1], cs[0])), cs[1]) + rows[2])
    return _tc_map(fn, out_rows, wn2.shape[1], [xn, agg, xs], [wn1g, wn2])


def _tc_head(x, p, out_rows):
    w1, b1 = p['W1'], p['b1'].reshape(1, -1)
    w2, b2 = p['W2'], p['b2'].reshape(1, -1)
    g, b = p['g'].reshape(1, -1), p['b'].reshape(1, -1)

    def fn(i, rows, cs):
        w1c, b1c, w2c, b2c, gc, bc = cs
        t = _dot(_leaky(_dot(rows[0], w1c) + b1c), w2c) + b2c
        mu = jnp.mean(t, axis=-1, keepdims=True)
        var = jnp.mean((t - mu) * (t - mu), axis=-1, keepdims=True)
        return (t - mu) * jax.lax.rsqrt(var + 1e-5) * gc + bc

    return _tc_map(fn, out_rows, 3, [x], [w1, b1, w2, b2, g, b])


def _tc_latent(z2, w1, b1, w2, b2col, blk):
    L = z2.shape[0]
    K = w1.shape[1]
    n = w2.shape[1]

    def body(z_ref, w1_ref, b1_ref, w2_ref, b2_ref, o_ref):
        a = _leaky(z_ref[...] * w1_ref[...] + b1_ref[...])
        o_ref[...] = (
            lax.dot_general(w2_ref[...], a, (((0,), (1,)), ((), ())),
                            preferred_element_type=jnp.float32)
            + b2_ref[...]
        )

    return pl.pallas_call(
        body,
        grid=(pl.cdiv(n, blk),),
        in_specs=[
            pl.BlockSpec((L, 1), lambda i: (0, 0)),
            pl.BlockSpec((1, K), lambda i: (0, 0)),
            pl.BlockSpec((1, K), lambda i: (0, 0)),
            pl.BlockSpec((K, blk), lambda i: (0, i)),
            pl.BlockSpec((blk, 1), lambda i: (i, 0)),
        ],
        out_specs=pl.BlockSpec((blk, L), lambda i: (i, 0)),
        out_shape=jax.ShapeDtypeStruct((n, L), jnp.float32),
    )(z2, w1, b1.reshape(1, K), w2, b2col)


def _sc_mesh():
    return plsc.VectorSubcoreMesh(core_axis_name="c", subcore_axis_name="s")


def _sc_params(layout_passes=False):
    cp = pltpu.CompilerParams()
    fields = pltpu.CompilerParams.__dataclass_fields__
    if not layout_passes and "needs_layout_passes" in fields:
        cp = dataclasses.replace(cp, needs_layout_passes=False)
    if "use_tc_tiling_on_sc" in fields:
        cp = dataclasses.replace(cp, use_tc_tiling_on_sc=False)
    return cp


def _stride_chunks(wid, nworkers, nchunks, do):
    per = (nchunks + nworkers - 1) // nworkers

    @pl.loop(0, per)
    def _(i):
        ci = wid + i * nworkers

        @pl.when(ci < nchunks)
        def _():
            do(ci)


def _sc_translate(table, idx):
    ep = idx.shape[0]
    t = table.shape[0]
    assert ep % _CHUNK == 0
    nchunks = ep // _CHUNK

    @functools.partial(
        pl.kernel,
        out_type=jax.ShapeDtypeStruct((ep,), jnp.int32),
        mesh=_sc_mesh(),
        compiler_params=_sc_params(),
        scratch_types=[
            pltpu.VMEM((t,), jnp.int32),
            pltpu.VMEM((_CHUNK,), jnp.int32),
            pltpu.VMEM((_CHUNK,), jnp.int32),
        ],
    )
    def k(idx_hbm, tab_hbm, out_hbm, tab_v, in_v, out_v):
        wid = lax.axis_index("s") * NC + lax.axis_index("c")
        pltpu.sync_copy(tab_hbm, tab_v)

        def do(ci):
            base = ci * _CHUNK
            pltpu.sync_copy(idx_hbm.at[pl.ds(base, _CHUNK)], in_v)

            @pl.loop(0, _CHUNK, step=NLANE)
            def _(j):
                out_v[pl.ds(j, NLANE)] = plsc.load_gather(
                    tab_v, [in_v[pl.ds(j, NLANE)]])

            pltpu.sync_copy(out_v, out_hbm.at[pl.ds(base, _CHUNK)])

        _stride_chunks(wid, NW, nchunks, do)

    return k(idx, table)


def _sc_gather(table, idx):
    w = table.shape[1]
    ep = idx.shape[0]
    assert ep % _CHUNK == 0
    idx2 = idx.reshape(1, ep)

    @functools.partial(
        pl.kernel,
        out_type=jax.ShapeDtypeStruct((ep, w), jnp.float32),
        mesh=_sc_mesh(),
        compiler_params=_sc_params(layout_passes=True),
    )
    def k(tab_hbm, idx_hbm, out_hbm):
        def body(i_vmem, o_vmem):
            pltpu.sync_copy(tab_hbm.at[i_vmem.at[0]], o_vmem)

        pltpu.emit_pipeline(
            body,
            grid=(ep // _CHUNK,),
            in_specs=[pl.BlockSpec((1, _CHUNK), index_map=lambda i: (0, i))],
            out_specs=[pl.BlockSpec((_CHUNK, w), index_map=lambda i: (i, 0))],
            core_axis_name=("c", "s"),
            dimension_semantics=(pltpu.PARALLEL,),
        )(idx_hbm, out_hbm)

    return k(table, idx2)


def _vextract(ref, k):
    g = (k // NLANE) * NLANE
    v = ref[pl.ds(g, NLANE)]
    return jnp.sum(jnp.where(lax.iota(jnp.int32, NLANE) == (k - g), v, 0))


def _segsum_meta(dst_pad, n_out):
    ep = dst_pad.shape[0]
    cap = _rup(-(-n_out // NW), 8)
    ds_sorted, order = lax.sort(
        [dst_pad, jnp.arange(ep, dtype=jnp.int32)], num_keys=1)
    bounds = jnp.arange(NW + 1, dtype=jnp.int32) * cap
    elo = jnp.searchsorted(ds_sorted, bounds).astype(jnp.int32)
    c0 = elo[:NW] // _BIG
    nch = -(-(elo[1:] - c0 * _BIG) // _BIG)
    pad16 = _rup(NW, NLANE)
    c0 = jnp.pad(c0, (0, pad16 - NW))
    nch = jnp.pad(nch, (0, pad16 - NW))
    return {'order': order, 'ds': ds_sorted, 'c0': c0, 'nch': nch, 'cap': cap}


def _sc_segsum(rows, meta):
    ep, w = rows.shape
    assert ep % _CHUNK == 0
    cap = meta['cap']
    nmeta = meta['c0'].shape[0]
    rows_sorted = _sc_gather(rows, meta['order'])
    rflat = rows_sorted.reshape(ep * w)
    zeros = jnp.zeros((_BIG * w,), jnp.float32)
    acc_n = (cap + 8) * w
    zc_full, zc_tail = acc_n // (_BIG * w), acc_n % (_BIG * w)

    @functools.partial(
        pl.kernel,
        out_type=jax.ShapeDtypeStruct((NW * cap * w,), jnp.float32),
        mesh=_sc_mesh(),
        compiler_params=_sc_params(),
        scratch_types=[
            pltpu.VMEM((acc_n,), jnp.float32),
            pltpu.VMEM((nmeta,), jnp.int32),
            pltpu.VMEM((nmeta,), jnp.int32),
            pltpu.VMEM((_BIG,), jnp.int32),
            pltpu.VMEM((_BIG * w,), jnp.float32),
            pltpu.SemaphoreType.DMA,
        ],
    )
    def k(rf_hbm, ds_hbm, c0_hbm, nch_hbm, zero_hbm, out_hbm,
          acc, c0v, nchv, didx, rv, sem):
        t = lax.axis_index("s") * NC + lax.axis_index("c")
        nlo = t * cap
        pltpu.sync_copy(c0_hbm, c0v)
        pltpu.sync_copy(nch_hbm, nchv)
        e0 = _vextract(c0v, t)
        nck = _vextract(nchv, t)

        for zi in range(zc_full):
            pltpu.sync_copy(zero_hbm, acc.at[pl.ds(zi * _BIG * w, _BIG * w)])
        if zc_tail:
            pltpu.sync_copy(zero_hbm.at[pl.ds(0, zc_tail)],
                            acc.at[pl.ds(zc_full * _BIG * w, zc_tail)])

        @pl.loop(0, nck)
        def _(i):
            base = (e0 + i) * _BIG
            h1 = pltpu.async_copy(ds_hbm.at[pl.ds(base, _BIG)], didx, sem)
            h2 = pltpu.async_copy(rf_hbm.at[pl.ds(base * w, _BIG * w)], rv, sem)
            h1.wait()
            h2.wait()

            @pl.loop(0, _BIG, step=NLANE)
            def _(j):
                d = didx[pl.ds(j, NLANE)] - nlo
                ok = (d >= 0) & (d < cap)
                didx[pl.ds(j, NLANE)] = jnp.where(ok, d, cap)

            @pl.loop(0, _BIG)
            def _(r):
                d = _vextract(didx, r)
                for c in range(0, w, NLANE):
                    acc[pl.ds(d * w + c, NLANE)] = (
                        acc[pl.ds(d * w + c, NLANE)]
                        + rv[pl.ds(r * w + c, NLANE)])

        cw = cap * w
        oc_full, oc_tail = cw // (_BIG * w), cw % (_BIG * w)
        for oi in range(oc_full):
            pltpu.sync_copy(acc.at[pl.ds(oi * _BIG * w, _BIG * w)],
                            out_hbm.at[pl.ds(nlo * w + oi * _BIG * w, _BIG * w)])
        if oc_tail:
            pltpu.sync_copy(acc.at[pl.ds(oc_full * _BIG * w, oc_tail)],
                            out_hbm.at[pl.ds(nlo * w + oc_full * _BIG * w, oc_tail)])

    out = k(rflat, meta['ds'], meta['c0'], meta['nch'], zeros)
    return out.reshape(NW * cap, w)


def _split_mpl(p, din):
    we1, wn1 = p['We1'], p['Wn1']
    return (we1[:din], we1[din:2 * din], we1[2 * din:],
            wn1[:din], wn1[din:], p['We2'], p['Wn2'])


def _mpl_direct(x, srcp, dstp, meta, ea, p, din, n):
    ws, wd, wa, wn1x, wn1g, we2, wn2 = _split_mpl(p, din)
    ps = _proj_pad(x, ws)
    pd = _proj_pad(x, wd)
    g1 = _sc_gather(ps, srcp)
    g2 = _sc_gather(pd, dstp)
    e_out = _tc_edge(g1, g2, ea, wa, we2, g1.shape[0])
    agg = _sc_segsum(e_out, meta)
    xo = _tc_node_direct(x, agg, wn1x, wn1g, wn2)
    return xo, e_out


def _res_up(x, ea, srcp_c, dstp_c, meta_c, srcp_f, dstp_f, meta_f,
            minv, einv, n_c, n_f, e_c, p, din1, dins):
    xm, eam = _mpl_direct(x, srcp_c, dstp_c, meta_c, ea, p['mpl1'], din1, n_c)
    dout1 = p['mpl1']['We2'].shape[1]

    fsrc = _sc_translate(minv, srcp_f)
    fdst = _sc_translate(minv, dstp_f)

    q = p['mpl2']
    ws2, wd2, wa2, wn1x2, wn1g2, we2_2, wn2_2 = _split_mpl(q, dout1)
    ps2 = _proj_pad(xm, ws2)
    pd2 = _proj_pad(xm, wd2)
    ep2 = _proj_pad(eam, wa2, n_valid=e_c)
    ga = _sc_gather(ps2, fsrc)
    gb = _sc_gather(pd2, fdst)
    gc = _sc_gather(ep2, einv)
    e2 = _tc_edge3(ga, gb, gc, we2_2, ga.shape[0])
    agg2 = _sc_segsum(e2, meta_f)
    xn2 = _sc_gather(_proj_pad(xm, wn1x2), minv)

    s = p['skip']
    wss, wds, was, wn1xs, wn1gs, we2s, wn2s = _split_mpl(s, dins)
    pss = _proj_pad(x, wss)
    pds = _proj_pad(x, wds)
    eps = _proj_pad(ea, was, n_valid=e_c)
    sa = _sc_gather(pss, fsrc)
    sb = _sc_gather(pds, fdst)
    sc = _sc_gather(eps, einv)
    gs = _tc_leaky3(sa, sb, sc, sa.shape[0])
    aggs = _sc_segsum(gs, meta_f)
    xns = _sc_gather(_proj_pad(x, wn1xs), minv)
    xs = _tc_node_skip(xns, aggs, we2s, wn1gs, wn2s, n_f)

    xo = _tc_node_main(xn2, agg2, xs, wn1g2, wn2_2, n_f)
    return xo, e2


def kernel(z, edge_index_l0, edge_index_l1, edge_index_l2, m_id_0, m_id_1,
           e_idx_0, e_idx_1, params):
    i32 = jnp.int32
    ei0 = edge_index_l0.astype(i32)
    ei1 = edge_index_l1.astype(i32)
    ei2 = edge_index_l2.astype(i32)

    e0p, e1p, e2p = _rup(E0, _BIG), _rup(E1, _BIG), _rup(E2, _BIG)
    n0p, n1p = _rup(N0, _CHUNK), _rup(N1, _CHUNK)

    def pad_idx(a, ep, fill):
        return jnp.pad(a, (0, ep - a.shape[0]), constant_values=fill)

    src0, dst0 = pad_idx(ei0[0], e0p, 0), pad_idx(ei0[1], e0p, 0)
    src1, dst1 = pad_idx(ei1[0], e1p, 0), pad_idx(ei1[1], e1p, 0)
    src2, dst2 = pad_idx(ei2[0], e2p, 0), pad_idx(ei2[1], e2p, 0)
    meta0 = _segsum_meta(pad_idx(ei0[1], e0p, N0), N0)
    meta1 = _segsum_meta(pad_idx(ei1[1], e1p, N1), N1)
    meta2 = _segsum_meta(pad_idx(ei2[1], e2p, N2), N2)

    minv0 = jnp.full((n0p,), N1, i32).at[m_id_0.astype(i32)].set(
        jnp.arange(N1, dtype=i32))
    minv1 = jnp.full((n1p,), N2, i32).at[m_id_1.astype(i32)].set(
        jnp.arange(N2, dtype=i32))
    einv0 = jnp.full((e0p,), E1, i32).at[e_idx_0.astype(i32)].set(
        jnp.arange(E1, dtype=i32))
    einv1 = jnp.full((e1p,), E2, i32).at[e_idx_1.astype(i32)].set(
        jnp.arange(E2, dtype=i32))

    z2 = z.reshape(LATENT, 1)
    px, pe = params['up_x'], params['up_e']
    x = _tc_latent(z2, px['W1'], px['b1'], px['W2'],
                   px['b2'].reshape(-1, 1), blk=512)
    e = _tc_latent(z2, pe['W1'], pe['b1'], pe['W2'],
                   pe['b2'].reshape(-1, 1), blk=1024)

    x, e = _mpl_direct(x, src2, dst2, meta2, e, params['bottom'], LATENT, N2)

    x, e = _res_up(x, e, src2, dst2, meta2, src1, dst1, meta1, minv1, einv1,
                   N2, N1, E2, params['l0'], 256, 256)

    x, e = _res_up(x, e, src1, dst1, meta1, src0, dst0, meta0, minv0, einv0,
                   N1, N0, E1, params['l1'], 128, 128)

    x, e = _mpl_direct(x, src0, dst0, meta0, e, params['final'], 64, N0)

    xo = _tc_head(x, params['out_n'], N0)
    eo = _tc_head(e[:E0], params['out_e'], E0)
    return xo, eo
